# baseline (speedup 1.0000x reference)
import functools

import jax
import jax.numpy as jnp
from jax import lax
from jax.experimental import pallas as pl
from jax.experimental.pallas import tpu as pltpu
from jax.experimental.pallas import tpu_sc as plsc

N_NODES = 10000
N_EDGES = 320000
IN_CH = 128
HID_CH = 128
OUT_CH = 64
N_GRAPHS = 64

NC = 2
NS = 16
NW = NC * NS

K_EDGE = 64
NCHUNK = 160
E_PER_TILE = K_EDGE * NCHUNK
E_PAD = NW * E_PER_TILE

ACC_ROWS = 10112
ZROWS_PER_TILE = ACC_ROWS // NS
HALF = NCHUNK // 2

CNT_ROWS = 10240


def _sc_mesh():
    return plsc.VectorSubcoreMesh(core_axis_name="c", subcore_axis_name="s",
                                  num_cores=NC, num_subcores=NS)


def _make_deg_kernel():
    @functools.partial(
        pl.kernel,
        out_type=jax.ShapeDtypeStruct((NW, CNT_ROWS), jnp.float32),
        mesh=_sc_mesh(),
        scratch_types=[
            pltpu.VMEM((E_PER_TILE,), jnp.int32),
            pltpu.VMEM((CNT_ROWS,), jnp.float32),
        ],
        compiler_params=pltpu.CompilerParams(needs_layout_passes=False),
    )
    def deg_kernel(dst_hbm, out_hbm, idx_v, cnt_v):
        c = lax.axis_index("c")
        s = lax.axis_index("s")
        wid = c * NS + s
        pltpu.sync_copy(dst_hbm.at[pl.ds(wid * E_PER_TILE, E_PER_TILE)], idx_v)

        zeros16 = jnp.zeros((16,), jnp.float32)
        ones16 = jnp.full((16,), 1.0, jnp.float32)

        def zero_body(i, _):
            cnt_v[pl.ds(i * 16, 16)] = zeros16
            return 0

        lax.fori_loop(0, CNT_ROWS // 16, zero_body, 0)

        def scat_body(i, _):
            idx = idx_v[pl.ds(i * 16, 16)]
            plsc.addupdate_scatter(cnt_v, [idx], ones16)
            return 0

        lax.fori_loop(0, E_PER_TILE // 16, scat_body, 0)
        pltpu.sync_copy(cnt_v, out_hbm.at[wid])

    return deg_kernel


def _make_scatter_kernel(width):
    nbuf = 4 if width == 128 else 8

    @functools.partial(
        pl.kernel,
        out_type=jax.ShapeDtypeStruct((NC, ACC_ROWS, width), jnp.float32),
        mesh=_sc_mesh(),
        scratch_types=[
            pltpu.VMEM((HALF, K_EDGE), jnp.int32),
            pltpu.VMEM((HALF, K_EDGE), jnp.int32),
            pltpu.VMEM((nbuf, K_EDGE, width), jnp.float32),
            pltpu.VMEM_SHARED((ACC_ROWS, width), jnp.float32),
            [pltpu.SemaphoreType.DMA] * nbuf,
            [pltpu.SemaphoreType.DMA] * nbuf,
        ],
        compiler_params=pltpu.CompilerParams(use_tc_tiling_on_sc=False),
    )
    def scatter_kernel(y_hbm, src_hbm, dst_hbm, zeros_hbm, out_hbm,
                       src_v, dst_v, rows_v, acc_sh, gsems, ssems):
        c = lax.axis_index("c")
        s = lax.axis_index("s")
        wid = c * NS + s

        zbase = s * ZROWS_PER_TILE
        pltpu.sync_copy(zeros_hbm, acc_sh.at[pl.ds(zbase, ZROWS_PER_TILE)])
        plsc.subcore_barrier()

        def wait_gather(b, j):
            pltpu.make_async_copy(
                y_hbm.at[src_v.at[j]], rows_v.at[b], gsems[b]).wait()

        def wait_scatter(b):
            pltpu.make_async_copy(
                rows_v.at[b], acc_sh.at[dst_v.at[0]], ssems[b]).wait()

        for half in range(2):
            pltpu.sync_copy(src_hbm.at[wid, pl.ds(half * HALF, HALF)], src_v)
            pltpu.sync_copy(dst_hbm.at[wid, pl.ds(half * HALF, HALF)], dst_v)

            for b in range(nbuf):
                pltpu.async_copy(y_hbm.at[src_v.at[b]], rows_v.at[b], gsems[b])

            def body(jj, _):
                base = jj * nbuf
                for b in range(nbuf):
                    wait_gather(b, base + b)
                    pltpu.async_copy(rows_v.at[b],
                                     acc_sh.at[dst_v.at[base + b]],
                                     ssems[b], add=True)
                for b in range(nbuf):
                    @pl.when(base + b + nbuf < HALF)
                    def _():
                        wait_scatter(b)
                        pltpu.async_copy(y_hbm.at[src_v.at[base + b + nbuf]],
                                         rows_v.at[b], gsems[b])
                return 0

            lax.fori_loop(0, HALF // nbuf, body, 0)
            for b in range(nbuf):
                wait_scatter(b)

        plsc.subcore_barrier()
        pltpu.sync_copy(acc_sh.at[pl.ds(zbase, ZROWS_PER_TILE)],
                        out_hbm.at[c, pl.ds(zbase, ZROWS_PER_TILE)])

    return scatter_kernel


_make_deg_kernel = functools.cache(_make_deg_kernel)
_make_scatter_kernel = functools.cache(_make_scatter_kernel)


def _tc1_body(x_ref, w1_ref, parts_ref, y1_ref, dinv_ref):
    ones = jnp.ones((NW, 1), jnp.float32)
    deg = lax.dot_general(parts_ref[...], ones,
                          (((0,), (0,)), ((), ())),
                          precision=lax.Precision.HIGHEST,
                          preferred_element_type=jnp.float32)
    deg = lax.slice(deg, (0, 0), (N_NODES, 1)) + 1.0
    dinv = lax.rsqrt(deg)
    xw = jnp.dot(x_ref[...], w1_ref[...],
                 precision=lax.Precision.HIGHEST,
                 preferred_element_type=jnp.float32)
    y1_ref[...] = xw * dinv
    dinv_ref[...] = dinv


def _tc2_body(p_ref, y1_ref, dinv_ref, b1_ref, w2_ref, y2_ref):
    dinv = dinv_ref[...]
    psum = p_ref[0] + p_ref[1]
    srow = lax.slice(psum, (0, 0), (N_NODES, HID_CH)) + y1_ref[...]
    h = jnp.maximum(srow * dinv + b1_ref[...], 0.0)
    y2_ref[...] = jnp.dot(h, w2_ref[...],
                          precision=lax.Precision.HIGHEST,
                          preferred_element_type=jnp.float32) * dinv


def _tc3_body(p_ref, y2_ref, dinv_ref, b2_ref, batch_ref, out_ref):
    psum = p_ref[0] + p_ref[1]
    srow = lax.slice(psum, (0, 0), (N_NODES, OUT_CH)) + y2_ref[...]
    h = srow * dinv_ref[...] + b2_ref[...]
    onehot = (batch_ref[...] == lax.broadcasted_iota(
        jnp.int32, (N_NODES, N_GRAPHS), 1)).astype(jnp.float32)
    seg = lax.dot_general(onehot, h, (((0,), (0,)), ((), ())),
                          precision=lax.Precision.HIGHEST,
                          preferred_element_type=jnp.float32)
    counts = lax.dot_general(onehot, jnp.ones((N_NODES, 1), jnp.float32),
                             (((0,), (0,)), ((), ())),
                             precision=lax.Precision.HIGHEST,
                             preferred_element_type=jnp.float32)
    out_ref[...] = seg / jnp.maximum(counts, 1.0)


_tc1 = pl.pallas_call(
    _tc1_body,
    out_shape=(jax.ShapeDtypeStruct((N_NODES, HID_CH), jnp.float32),
               jax.ShapeDtypeStruct((N_NODES, 1), jnp.float32)))

_tc2 = pl.pallas_call(
    _tc2_body,
    out_shape=jax.ShapeDtypeStruct((N_NODES, OUT_CH), jnp.float32))

_tc3 = pl.pallas_call(
    _tc3_body,
    out_shape=jax.ShapeDtypeStruct((N_GRAPHS, OUT_CH), jnp.float32))


def kernel(x, edge_index, batch, W1, b1, W2, b2):
    src = edge_index[0].astype(jnp.int32)
    dst = edge_index[1].astype(jnp.int32)
    pad = E_PAD - N_EDGES
    src_p = jnp.concatenate([src, jnp.zeros((pad,), jnp.int32)])
    pad_dst = N_NODES + jnp.arange(pad, dtype=jnp.int32) % (ACC_ROWS - N_NODES)
    dst_p = jnp.concatenate([dst, pad_dst])
    src3 = src_p.reshape(NW, NCHUNK, K_EDGE)
    dst3 = dst_p.reshape(NW, NCHUNK, K_EDGE)

    zeros_hid = jnp.zeros((ZROWS_PER_TILE, HID_CH), jnp.float32)
    zeros_out = jnp.zeros((ZROWS_PER_TILE, OUT_CH), jnp.float32)

    deg_kernel = _make_deg_kernel()
    scatter_hid = _make_scatter_kernel(HID_CH)
    scatter_out = _make_scatter_kernel(OUT_CH)

    deg_parts = deg_kernel(dst_p)
    y1, dinv = _tc1(x, W1, deg_parts)
    p1 = scatter_hid(y1, src3, dst3, zeros_hid)
    y2 = _tc2(p1, y1, dinv, b1.reshape(1, HID_CH), W2)
    p2 = scatter_out(y2, src3, dst3, zeros_out)
    return _tc3(p2, y2, dinv, b2.reshape(1, OUT_CH),
                batch.astype(jnp.int32).reshape(N_NODES, 1))


# speedup vs baseline: 1.0149x; 1.0149x over previous
import functools

import jax
import jax.numpy as jnp
from jax import lax
from jax.experimental import pallas as pl
from jax.experimental.pallas import tpu as pltpu
from jax.experimental.pallas import tpu_sc as plsc

N_NODES = 10000
N_EDGES = 320000
IN_CH = 128
HID_CH = 128
OUT_CH = 64
N_GRAPHS = 64

NC = 2
NS = 16
NW = NC * NS

K_EDGE = 64
NCHUNK = 160
E_PER_TILE = K_EDGE * NCHUNK
E_PAD = NW * E_PER_TILE

NCH_FAST = 256
NCH_SLOW = 64
FAST_CORE = 0
HCH = 128
E_ALLOC = NW * E_PER_TILE + NCH_SLOW * K_EDGE

ACC_ROWS = 10112
ZROWS_PER_TILE = ACC_ROWS // NS
HALF = NCHUNK // 2

CNT_ROWS = 10240


def _sc_mesh():
    return plsc.VectorSubcoreMesh(core_axis_name="c", subcore_axis_name="s",
                                  num_cores=NC, num_subcores=NS)


def _make_deg_kernel():
    @functools.partial(
        pl.kernel,
        out_type=jax.ShapeDtypeStruct((NW, CNT_ROWS), jnp.float32),
        mesh=_sc_mesh(),
        scratch_types=[
            pltpu.VMEM((E_PER_TILE,), jnp.int32),
            pltpu.VMEM((CNT_ROWS,), jnp.float32),
        ],
        compiler_params=pltpu.CompilerParams(needs_layout_passes=False),
    )
    def deg_kernel(dst_hbm, out_hbm, idx_v, cnt_v):
        c = lax.axis_index("c")
        s = lax.axis_index("s")
        wid = c * NS + s
        pltpu.sync_copy(dst_hbm.at[pl.ds(wid * E_PER_TILE, E_PER_TILE)], idx_v)

        zeros16 = jnp.zeros((16,), jnp.float32)
        ones16 = jnp.full((16,), 1.0, jnp.float32)

        def zero_body(i, _):
            cnt_v[pl.ds(i * 16, 16)] = zeros16
            return 0

        lax.fori_loop(0, CNT_ROWS // 16, zero_body, 0)

        def scat_body(i, _):
            idx = idx_v[pl.ds(i * 16, 16)]
            plsc.addupdate_scatter(cnt_v, [idx], ones16)
            return 0

        lax.fori_loop(0, E_PER_TILE // 16, scat_body, 0)
        pltpu.sync_copy(cnt_v, out_hbm.at[wid])

    return deg_kernel


def _make_scatter_kernel(width):
    nbuf = 4 if width == 128 else 8

    @functools.partial(
        pl.kernel,
        out_type=jax.ShapeDtypeStruct((NC, ACC_ROWS, width), jnp.float32),
        mesh=_sc_mesh(),
        scratch_types=[
            pltpu.VMEM((HCH, K_EDGE), jnp.int32),
            pltpu.VMEM((HCH, K_EDGE), jnp.int32),
            pltpu.VMEM((nbuf, K_EDGE, width), jnp.float32),
            pltpu.VMEM_SHARED((ACC_ROWS, width), jnp.float32),
            [pltpu.SemaphoreType.DMA] * nbuf,
            [pltpu.SemaphoreType.DMA] * nbuf,
        ],
        compiler_params=pltpu.CompilerParams(use_tc_tiling_on_sc=False),
    )
    def scatter_kernel(y_hbm, src_hbm, dst_hbm, zeros_hbm, out_hbm,
                       src_v, dst_v, rows_v, acc_sh, gsems, ssems):
        c = lax.axis_index("c")
        s = lax.axis_index("s")
        on_fast = c == FAST_CORE
        cbase = lax.select(on_fast, s * NCH_FAST,
                           NS * NCH_FAST + s * NCH_SLOW)
        nch = lax.select(on_fast, NCH_FAST, NCH_SLOW)

        zbase = s * ZROWS_PER_TILE
        pltpu.sync_copy(zeros_hbm, acc_sh.at[pl.ds(zbase, ZROWS_PER_TILE)])
        plsc.subcore_barrier()

        def wait_gather(b, j):
            pltpu.make_async_copy(
                y_hbm.at[src_v.at[j]], rows_v.at[b], gsems[b]).wait()

        def wait_scatter(b):
            pltpu.make_async_copy(
                rows_v.at[b], acc_sh.at[dst_v.at[0]], ssems[b]).wait()

        for half in range(2):
            start = half * HCH
            n_half = jnp.minimum(nch - start, HCH)

            @pl.when(n_half > 0)
            def _():
                pltpu.sync_copy(src_hbm.at[pl.ds(cbase + start, HCH)], src_v)
                pltpu.sync_copy(dst_hbm.at[pl.ds(cbase + start, HCH)], dst_v)

                for b in range(nbuf):
                    pltpu.async_copy(y_hbm.at[src_v.at[b]], rows_v.at[b],
                                     gsems[b])

                def body(jj, _):
                    base = jj * nbuf
                    for b in range(nbuf):
                        wait_gather(b, base + b)
                        pltpu.async_copy(rows_v.at[b],
                                         acc_sh.at[dst_v.at[base + b]],
                                         ssems[b], add=True)
                    for b in range(nbuf):
                        @pl.when(base + b + nbuf < n_half)
                        def _():
                            wait_scatter(b)
                            pltpu.async_copy(
                                y_hbm.at[src_v.at[base + b + nbuf]],
                                rows_v.at[b], gsems[b])
                    return 0

                lax.fori_loop(0, n_half // nbuf, body, 0)
                for b in range(nbuf):
                    wait_scatter(b)

        plsc.subcore_barrier()
        pltpu.sync_copy(acc_sh.at[pl.ds(zbase, ZROWS_PER_TILE)],
                        out_hbm.at[c, pl.ds(zbase, ZROWS_PER_TILE)])

    return scatter_kernel


_make_deg_kernel = functools.cache(_make_deg_kernel)
_make_scatter_kernel = functools.cache(_make_scatter_kernel)


def _tc1_body(x_ref, w1_ref, parts_ref, y1_ref, dinv_ref):
    ones = jnp.ones((NW, 1), jnp.float32)
    deg = lax.dot_general(parts_ref[...], ones,
                          (((0,), (0,)), ((), ())),
                          precision=lax.Precision.HIGHEST,
                          preferred_element_type=jnp.float32)
    deg = lax.slice(deg, (0, 0), (N_NODES, 1)) + 1.0
    dinv = lax.rsqrt(deg)
    xw = jnp.dot(x_ref[...], w1_ref[...],
                 precision=lax.Precision.HIGHEST,
                 preferred_element_type=jnp.float32)
    y1_ref[...] = xw * dinv
    dinv_ref[...] = dinv


def _tc2_body(p_ref, y1_ref, dinv_ref, b1_ref, w2_ref, y2_ref):
    dinv = dinv_ref[...]
    psum = p_ref[0] + p_ref[1]
    srow = lax.slice(psum, (0, 0), (N_NODES, HID_CH)) + y1_ref[...]
    h = jnp.maximum(srow * dinv + b1_ref[...], 0.0)
    y2_ref[...] = jnp.dot(h, w2_ref[...],
                          precision=lax.Precision.HIGHEST,
                          preferred_element_type=jnp.float32) * dinv


def _tc3_body(p_ref, y2_ref, dinv_ref, b2_ref, batch_ref, out_ref):
    psum = p_ref[0] + p_ref[1]
    srow = lax.slice(psum, (0, 0), (N_NODES, OUT_CH)) + y2_ref[...]
    h = srow * dinv_ref[...] + b2_ref[...]
    onehot = (batch_ref[...] == lax.broadcasted_iota(
        jnp.int32, (N_NODES, N_GRAPHS), 1)).astype(jnp.float32)
    seg = lax.dot_general(onehot, h, (((0,), (0,)), ((), ())),
                          precision=lax.Precision.HIGHEST,
                          preferred_element_type=jnp.float32)
    counts = lax.dot_general(onehot, jnp.ones((N_NODES, 1), jnp.float32),
                             (((0,), (0,)), ((), ())),
                             precision=lax.Precision.HIGHEST,
                             preferred_element_type=jnp.float32)
    out_ref[...] = seg / jnp.maximum(counts, 1.0)


_tc1 = pl.pallas_call(
    _tc1_body,
    out_shape=(jax.ShapeDtypeStruct((N_NODES, HID_CH), jnp.float32),
               jax.ShapeDtypeStruct((N_NODES, 1), jnp.float32)))

_tc2 = pl.pallas_call(
    _tc2_body,
    out_shape=jax.ShapeDtypeStruct((N_NODES, OUT_CH), jnp.float32))

_tc3 = pl.pallas_call(
    _tc3_body,
    out_shape=jax.ShapeDtypeStruct((N_GRAPHS, OUT_CH), jnp.float32))


def kernel(x, edge_index, batch, W1, b1, W2, b2):
    src = edge_index[0].astype(jnp.int32)
    dst = edge_index[1].astype(jnp.int32)
    pad = E_ALLOC - N_EDGES
    src_p = jnp.concatenate([src, jnp.zeros((pad,), jnp.int32)])
    pad_dst = N_NODES + jnp.arange(pad, dtype=jnp.int32) % (ACC_ROWS - N_NODES)
    dst_p = jnp.concatenate([dst, pad_dst])
    src3 = src_p.reshape(-1, K_EDGE)
    dst3 = dst_p.reshape(-1, K_EDGE)

    zeros_hid = jnp.zeros((ZROWS_PER_TILE, HID_CH), jnp.float32)
    zeros_out = jnp.zeros((ZROWS_PER_TILE, OUT_CH), jnp.float32)

    deg_kernel = _make_deg_kernel()
    scatter_hid = _make_scatter_kernel(HID_CH)
    scatter_out = _make_scatter_kernel(OUT_CH)

    deg_parts = deg_kernel(dst_p[:E_PAD])
    y1, dinv = _tc1(x, W1, deg_parts)
    p1 = scatter_hid(y1, src3, dst3, zeros_hid)
    y2 = _tc2(p1, y1, dinv, b1.reshape(1, HID_CH), W2)
    p2 = scatter_out(y2, src3, dst3, zeros_out)
    return _tc3(p2, y2, dinv, b2.reshape(1, OUT_CH),
                batch.astype(jnp.int32).reshape(N_NODES, 1))


# speedup vs baseline: 2.9092x; 2.8666x over previous
import functools

import jax
import jax.numpy as jnp
from jax import lax
from jax.experimental import pallas as pl
from jax.experimental.pallas import tpu as pltpu
from jax.experimental.pallas import tpu_sc as plsc

N_NODES = 10000
N_EDGES = 320000
IN_CH = 128
HID_CH = 128
OUT_CH = 64
N_GRAPHS = 64

NC = 2
NS = 16
NW = NC * NS

K_EDGE = 64
NCHUNK = 160
E_PER_TILE = K_EDGE * NCHUNK
E_PAD = NW * E_PER_TILE

HCH = 128
REAL_PER_TILE = N_EDGES // NW
PAD_PER_TILE = E_PER_TILE - REAL_PER_TILE

ACC_ROWS = 10112
ZROWS_PER_TILE = ACC_ROWS // NS
HALF = NCHUNK // 2

CNT_ROWS = 10240


def _sc_mesh():
    return plsc.VectorSubcoreMesh(core_axis_name="c", subcore_axis_name="s",
                                  num_cores=NC, num_subcores=NS)


def _make_deg_kernel():
    @functools.partial(
        pl.kernel,
        out_type=jax.ShapeDtypeStruct((NW, CNT_ROWS), jnp.float32),
        mesh=_sc_mesh(),
        scratch_types=[
            pltpu.VMEM((E_PER_TILE,), jnp.int32),
            pltpu.VMEM((CNT_ROWS,), jnp.float32),
        ],
        compiler_params=pltpu.CompilerParams(needs_layout_passes=False),
    )
    def deg_kernel(dst_hbm, out_hbm, idx_v, cnt_v):
        c = lax.axis_index("c")
        s = lax.axis_index("s")
        wid = c * NS + s
        pltpu.sync_copy(dst_hbm.at[pl.ds(wid * E_PER_TILE, E_PER_TILE)], idx_v)

        zeros16 = jnp.zeros((16,), jnp.float32)
        ones16 = jnp.full((16,), 1.0, jnp.float32)

        def zero_body(i, _):
            cnt_v[pl.ds(i * 16, 16)] = zeros16
            return 0

        lax.fori_loop(0, CNT_ROWS // 16, zero_body, 0)

        def scat_body(i, _):
            idx = idx_v[pl.ds(i * 16, 16)]
            plsc.addupdate_scatter(cnt_v, [idx], ones16)
            return 0

        lax.fori_loop(0, REAL_PER_TILE // 16, scat_body, 0)
        pltpu.sync_copy(cnt_v, out_hbm.at[wid])

    return deg_kernel


def _make_scatter_kernel(width):
    nbuf = 4 if width == 128 else 8

    @functools.partial(
        pl.kernel,
        out_type=jax.ShapeDtypeStruct((NC, ACC_ROWS, width), jnp.float32),
        mesh=_sc_mesh(),
        scratch_types=[
            pltpu.VMEM((HCH, K_EDGE), jnp.int32),
            pltpu.VMEM((HCH, K_EDGE), jnp.int32),
            pltpu.VMEM((nbuf, K_EDGE, width), jnp.float32),
            pltpu.VMEM_SHARED((ACC_ROWS, width), jnp.float32),
            [pltpu.SemaphoreType.DMA] * nbuf,
            [pltpu.SemaphoreType.DMA] * nbuf,
        ],
        compiler_params=pltpu.CompilerParams(use_tc_tiling_on_sc=False),
    )
    def scatter_kernel(y_hbm, src_hbm, dst_hbm, zeros_hbm, out_hbm,
                       src_v, dst_v, rows_v, acc_sh, gsems, ssems):
        c = lax.axis_index("c")
        s = lax.axis_index("s")
        wid = c * NS + s
        cbase = wid * NCHUNK

        zbase = s * ZROWS_PER_TILE
        pltpu.sync_copy(zeros_hbm, acc_sh.at[pl.ds(zbase, ZROWS_PER_TILE)])
        plsc.subcore_barrier()

        def wait_gather(b, j):
            pltpu.make_async_copy(
                y_hbm.at[src_v.at[j]], rows_v.at[b], gsems[b]).wait()

        def wait_scatter(b):
            pltpu.make_async_copy(
                rows_v.at[b], acc_sh.at[dst_v.at[0]], ssems[b]).wait()

        for half in range(2):
            start = half * HCH
            n_half = min(NCHUNK - start, HCH)

            pltpu.sync_copy(src_hbm.at[pl.ds(cbase + start, n_half)],
                            src_v.at[pl.ds(0, n_half)])
            pltpu.sync_copy(dst_hbm.at[pl.ds(cbase + start, n_half)],
                            dst_v.at[pl.ds(0, n_half)])

            for b in range(nbuf):
                pltpu.async_copy(y_hbm.at[src_v.at[b]], rows_v.at[b],
                                 gsems[b])

            def body(jj, _):
                base = jj * nbuf
                for b in range(nbuf):
                    wait_gather(b, base + b)
                    pltpu.async_copy(rows_v.at[b],
                                     acc_sh.at[dst_v.at[base + b]],
                                     ssems[b], add=True)
                for b in range(nbuf):
                    @pl.when(base + b + nbuf < n_half)
                    def _():
                        wait_scatter(b)
                        pltpu.async_copy(
                            y_hbm.at[src_v.at[base + b + nbuf]],
                            rows_v.at[b], gsems[b])
                return 0

            lax.fori_loop(0, n_half // nbuf, body, 0)
            for b in range(nbuf):
                wait_scatter(b)

        plsc.subcore_barrier()
        pltpu.sync_copy(acc_sh.at[pl.ds(zbase, ZROWS_PER_TILE)],
                        out_hbm.at[c, pl.ds(zbase, ZROWS_PER_TILE)])

    return scatter_kernel


_make_deg_kernel = functools.cache(_make_deg_kernel)
_make_scatter_kernel = functools.cache(_make_scatter_kernel)


def _tc1_body(x_ref, w1_ref, parts_ref, y1_ref, dinv_ref):
    ones = jnp.ones((NW, 1), jnp.float32)
    deg = lax.dot_general(parts_ref[...], ones,
                          (((0,), (0,)), ((), ())),
                          precision=lax.Precision.HIGHEST,
                          preferred_element_type=jnp.float32)
    deg = lax.slice(deg, (0, 0), (N_NODES, 1)) + 1.0
    dinv = lax.rsqrt(deg)
    xw = jnp.dot(x_ref[...], w1_ref[...],
                 precision=lax.Precision.HIGHEST,
                 preferred_element_type=jnp.float32)
    y1_ref[pl.ds(0, N_NODES), :] = xw * dinv
    y1_ref[pl.ds(N_NODES, ACC_ROWS - N_NODES), :] = jnp.zeros(
        (ACC_ROWS - N_NODES, HID_CH), jnp.float32)
    dinv_ref[...] = dinv


def _tc2_body(p_ref, y1_ref, dinv_ref, b1_ref, w2_ref, y2_ref):
    dinv = dinv_ref[...]
    psum = p_ref[0] + p_ref[1]
    srow = (lax.slice(psum, (0, 0), (N_NODES, HID_CH))
            + lax.slice(y1_ref[...], (0, 0), (N_NODES, HID_CH)))
    h = jnp.maximum(srow * dinv + b1_ref[...], 0.0)
    y2_ref[pl.ds(0, N_NODES), :] = jnp.dot(
        h, w2_ref[...], precision=lax.Precision.HIGHEST,
        preferred_element_type=jnp.float32) * dinv
    y2_ref[pl.ds(N_NODES, ACC_ROWS - N_NODES), :] = jnp.zeros(
        (ACC_ROWS - N_NODES, OUT_CH), jnp.float32)


def _tc3_body(p_ref, y2_ref, dinv_ref, b2_ref, batch_ref, out_ref):
    psum = p_ref[0] + p_ref[1]
    srow = (lax.slice(psum, (0, 0), (N_NODES, OUT_CH))
            + lax.slice(y2_ref[...], (0, 0), (N_NODES, OUT_CH)))
    h = srow * dinv_ref[...] + b2_ref[...]
    onehot = (batch_ref[...] == lax.broadcasted_iota(
        jnp.int32, (N_NODES, N_GRAPHS), 1)).astype(jnp.float32)
    seg = lax.dot_general(onehot, h, (((0,), (0,)), ((), ())),
                          precision=lax.Precision.HIGHEST,
                          preferred_element_type=jnp.float32)
    counts = lax.dot_general(onehot, jnp.ones((N_NODES, 1), jnp.float32),
                             (((0,), (0,)), ((), ())),
                             precision=lax.Precision.HIGHEST,
                             preferred_element_type=jnp.float32)
    out_ref[...] = seg / jnp.maximum(counts, 1.0)


_tc1 = pl.pallas_call(
    _tc1_body,
    out_shape=(jax.ShapeDtypeStruct((ACC_ROWS, HID_CH), jnp.float32),
               jax.ShapeDtypeStruct((N_NODES, 1), jnp.float32)))

_tc2 = pl.pallas_call(
    _tc2_body,
    out_shape=jax.ShapeDtypeStruct((ACC_ROWS, OUT_CH), jnp.float32))

_tc3 = pl.pallas_call(
    _tc3_body,
    out_shape=jax.ShapeDtypeStruct((N_GRAPHS, OUT_CH), jnp.float32))


def kernel(x, edge_index, batch, W1, b1, W2, b2):
    src = edge_index[0].astype(jnp.int32)
    dst = edge_index[1].astype(jnp.int32)
    i_pad = jnp.arange(PAD_PER_TILE, dtype=jnp.int32)[None, :]
    t_pad = jnp.arange(NW, dtype=jnp.int32)[:, None]
    pad_src = N_NODES + (i_pad + t_pad) % (ACC_ROWS - N_NODES)
    pad_dst = (i_pad * 37 + t_pad * 977) % N_NODES
    src_p = jnp.concatenate(
        [src.reshape(NW, REAL_PER_TILE), pad_src], axis=1).reshape(-1)
    dst_p = jnp.concatenate(
        [dst.reshape(NW, REAL_PER_TILE), pad_dst], axis=1).reshape(-1)
    src3 = src_p.reshape(-1, K_EDGE)
    dst3 = dst_p.reshape(-1, K_EDGE)

    zeros_hid = jnp.zeros((ZROWS_PER_TILE, HID_CH), jnp.float32)
    zeros_out = jnp.zeros((ZROWS_PER_TILE, OUT_CH), jnp.float32)

    deg_kernel = _make_deg_kernel()
    scatter_hid = _make_scatter_kernel(HID_CH)
    scatter_out = _make_scatter_kernel(OUT_CH)

    deg_parts = deg_kernel(dst_p)
    y1, dinv = _tc1(x, W1, deg_parts)
    p1 = scatter_hid(y1, src3, dst3, zeros_hid)
    y2 = _tc2(p1, y1, dinv, b1.reshape(1, HID_CH), W2)
    p2 = scatter_out(y2, src3, dst3, zeros_out)
    return _tc3(p2, y2, dinv, b2.reshape(1, OUT_CH),
                batch.astype(jnp.int32).reshape(N_NODES, 1))
